# baseline (speedup 1.0000x reference)
import jax
import jax.numpy as jnp
from jax.experimental import pallas as pl
from jax.experimental.pallas import tpu as pltpu

_CH = 40
_NBUF = 8


def _gcn_kernel(x_ref, w_ref, b_ref, adj_ref, out_ref, support_ref, bufs_ref, sems):
    n = x_ref.shape[0]
    nchunks = n // _CH

    def chunk_copy(i, slot):
        return pltpu.make_async_copy(
            adj_ref.at[pl.ds(i * _CH, _CH), :],
            bufs_ref.at[slot],
            sems.at[slot],
        )

    for s in range(_NBUF):
        chunk_copy(s, s).start()

    support_ref[...] = jnp.dot(
        x_ref[...], w_ref[...], preferred_element_type=jnp.float32
    )

    def loop_body(i, carry):
        slot = jax.lax.rem(i, _NBUF)
        chunk_copy(i, slot).wait()
        out_ref[pl.ds(i * _CH, _CH), :] = (
            jnp.dot(bufs_ref[slot], support_ref[...], preferred_element_type=jnp.float32)
            + b_ref[...]
        )
        nxt = i + _NBUF

        @pl.when(nxt < nchunks)
        def _():
            chunk_copy(nxt, slot).start()

        return carry

    jax.lax.fori_loop(0, nchunks, loop_body, 0)


def kernel(x, adj, W, b):
    n, din = x.shape
    dout = W.shape[1]
    b2 = b.reshape(1, dout)
    return pl.pallas_call(
        _gcn_kernel,
        in_specs=[
            pl.BlockSpec((n, din), lambda: (0, 0)),
            pl.BlockSpec((din, dout), lambda: (0, 0)),
            pl.BlockSpec((1, dout), lambda: (0, 0)),
            pl.BlockSpec(memory_space=pl.ANY),
        ],
        out_specs=pl.BlockSpec((n, dout), lambda: (0, 0)),
        out_shape=jax.ShapeDtypeStruct((n, dout), jnp.float32),
        scratch_shapes=[
            pltpu.VMEM((n, dout), jnp.float32),
            pltpu.VMEM((_NBUF, _CH, n), jnp.float32),
            pltpu.SemaphoreType.DMA((_NBUF,)),
        ],
    )(x, W, b2, adj)


# speedup vs baseline: 1.1759x; 1.1759x over previous
import jax
import jax.numpy as jnp
from jax.experimental import pallas as pl
from jax.experimental.pallas import tpu as pltpu

_CH = 80
_NBUF = 8


def _gcn_kernel(x_ref, w_ref, b_ref, adj_ref, out_ref, support_ref, bufs_ref, sems):
    n = x_ref.shape[0]
    nchunks = n // _CH

    def chunk_copy(i, slot):
        return pltpu.make_async_copy(
            adj_ref.at[pl.ds(i * _CH, _CH), :],
            bufs_ref.at[slot],
            sems.at[slot],
        )

    for s in range(_NBUF):
        chunk_copy(s, s).start()

    support_ref[...] = jnp.dot(
        x_ref[...], w_ref[...], preferred_element_type=jnp.float32
    )

    def loop_body(i, carry):
        slot = jax.lax.rem(i, _NBUF)
        chunk_copy(i, slot).wait()
        out_ref[pl.ds(i * _CH, _CH), :] = (
            jnp.dot(bufs_ref[slot], support_ref[...], preferred_element_type=jnp.float32)
            + b_ref[...]
        )
        nxt = i + _NBUF

        @pl.when(nxt < nchunks)
        def _():
            chunk_copy(nxt, slot).start()

        return carry

    jax.lax.fori_loop(0, nchunks, loop_body, 0)


def kernel(x, adj, W, b):
    n, din = x.shape
    dout = W.shape[1]
    b2 = b.reshape(1, dout)
    return pl.pallas_call(
        _gcn_kernel,
        in_specs=[
            pl.BlockSpec((n, din), lambda: (0, 0)),
            pl.BlockSpec((din, dout), lambda: (0, 0)),
            pl.BlockSpec((1, dout), lambda: (0, 0)),
            pl.BlockSpec(memory_space=pl.ANY),
        ],
        out_specs=pl.BlockSpec((n, dout), lambda: (0, 0)),
        out_shape=jax.ShapeDtypeStruct((n, dout), jnp.float32),
        scratch_shapes=[
            pltpu.VMEM((n, dout), jnp.float32),
            pltpu.VMEM((_NBUF, _CH, n), jnp.float32),
            pltpu.SemaphoreType.DMA((_NBUF,)),
        ],
    )(x, W, b2, adj)


# speedup vs baseline: 1.1765x; 1.0005x over previous
import jax
import jax.numpy as jnp
from jax.experimental import pallas as pl
from jax.experimental.pallas import tpu as pltpu

_CH = 80
_NBUF = 8


def _gcn_kernel(x_ref, w_ref, b_ref, adj_ref, out_ref, support_ref, bufs_ref, sems):
    n = x_ref.shape[0]
    nchunks = n // _CH

    def chunk_copy(i, slot):
        return pltpu.make_async_copy(
            adj_ref.at[pl.ds(i * _CH, _CH), :],
            bufs_ref.at[slot],
            sems.at[slot],
        )

    for s in range(_NBUF):
        chunk_copy(s, s).start()

    support_ref[...] = jnp.dot(
        x_ref[...], w_ref[...], preferred_element_type=jnp.float32
    ).astype(jnp.bfloat16)

    def loop_body(i, carry):
        slot = jax.lax.rem(i, _NBUF)
        chunk_copy(i, slot).wait()
        out_ref[pl.ds(i * _CH, _CH), :] = (
            jnp.dot(
                bufs_ref[slot].astype(jnp.bfloat16),
                support_ref[...],
                preferred_element_type=jnp.float32,
            )
            + b_ref[...]
        )
        nxt = i + _NBUF

        @pl.when(nxt < nchunks)
        def _():
            chunk_copy(nxt, slot).start()

        return carry

    jax.lax.fori_loop(0, nchunks, loop_body, 0)


def kernel(x, adj, W, b):
    n, din = x.shape
    dout = W.shape[1]
    b2 = b.reshape(1, dout)
    return pl.pallas_call(
        _gcn_kernel,
        in_specs=[
            pl.BlockSpec((n, din), lambda: (0, 0)),
            pl.BlockSpec((din, dout), lambda: (0, 0)),
            pl.BlockSpec((1, dout), lambda: (0, 0)),
            pl.BlockSpec(memory_space=pl.ANY),
        ],
        out_specs=pl.BlockSpec((n, dout), lambda: (0, 0)),
        out_shape=jax.ShapeDtypeStruct((n, dout), jnp.float32),
        scratch_shapes=[
            pltpu.VMEM((n, dout), jnp.bfloat16),
            pltpu.VMEM((_NBUF, _CH, n), jnp.float32),
            pltpu.SemaphoreType.DMA((_NBUF,)),
        ],
    )(x, W, b2, adj)


# speedup vs baseline: 1.1879x; 1.0096x over previous
import jax
import jax.numpy as jnp
from jax.experimental import pallas as pl
from jax.experimental.pallas import tpu as pltpu


def _gcn_kernel(x_ref, w_ref, b_ref, adj_ref, out_ref, support_ref):
    @pl.when(pl.program_id(0) == 0)
    def _():
        support_ref[...] = jnp.dot(
            x_ref[...], w_ref[...], preferred_element_type=jnp.float32
        )

    out_ref[...] = (
        jnp.dot(adj_ref[...], support_ref[...], preferred_element_type=jnp.float32)
        + b_ref[...]
    )


def kernel(x, adj, W, b):
    n, din = x.shape
    dout = W.shape[1]
    bm = 400
    b2 = b.reshape(1, dout)
    return pl.pallas_call(
        _gcn_kernel,
        grid=(n // bm,),
        in_specs=[
            pl.BlockSpec((n, din), lambda m: (0, 0)),
            pl.BlockSpec((din, dout), lambda m: (0, 0)),
            pl.BlockSpec((1, dout), lambda m: (0, 0)),
            pl.BlockSpec((bm, n), lambda m: (m, 0)),
        ],
        out_specs=pl.BlockSpec((bm, dout), lambda m: (m, 0)),
        out_shape=jax.ShapeDtypeStruct((n, dout), jnp.float32),
        scratch_shapes=[pltpu.VMEM((n, dout), jnp.float32)],
    )(x, W, b2, adj)
